# Initial kernel scaffold; baseline (speedup 1.0000x reference)
import jax
import jax.numpy as jnp
from jax.experimental import pallas as pl


def kernel(x, pred):
    raise NotImplementedError("write your pallas kernel here")


# speedup vs baseline: 10.9695x; 10.9695x over previous
import functools

import jax
import jax.numpy as jnp
from jax import lax
from jax.experimental import pallas as pl
from jax.experimental.pallas import tpu as pltpu

_CONF = 0.25
_IOU_THR = 0.45
_MAX_DET = 300
_LANES = 128
_MD_PAD = 304


def _nms_body(pred_ref, out_ref, s_ref, ox1_ref, oy1_ref, ox2_ref, oy2_ref,
              nx1_ref, ny1_ref, nx2_ref, ny2_ref, cls_ref, a2_ref,
              *, img_max, nc, rows):
    score = pred_ref[0, 4]
    for c in range(1, nc):
        score = jnp.maximum(score, pred_ref[0, 4 + c])
    clsi = jnp.full((rows, _LANES), nc, jnp.int32)
    for c in range(nc):
        clsi = jnp.minimum(clsi, jnp.where(pred_ref[0, 4 + c] == score, c, nc))
    clsf = clsi.astype(jnp.float32)

    bx = pred_ref[0, 0]
    by = pred_ref[0, 1]
    hw = pred_ref[0, 2] / 2.0
    hh = pred_ref[0, 3] / 2.0
    x1 = bx - hw
    y1 = by - hh
    x2 = bx + hw
    y2 = by + hh
    nx1 = x1 / img_max + clsf
    ny1 = y1 / img_max + clsf
    nx2 = x2 / img_max + clsf
    ny2 = y2 / img_max + clsf
    a2 = jnp.maximum(nx2 - nx1, 0.0) * jnp.maximum(ny2 - ny1, 0.0)

    s_ref[...] = jnp.where(score > _CONF, score, -jnp.inf)
    ox1_ref[...] = x1
    oy1_ref[...] = y1
    ox2_ref[...] = x2
    oy2_ref[...] = y2
    nx1_ref[...] = nx1
    ny1_ref[...] = ny1
    nx2_ref[...] = nx2
    ny2_ref[...] = ny2
    cls_ref[...] = clsf
    a2_ref[...] = a2

    iota_r = lax.broadcasted_iota(jnp.int32, (rows, _LANES), 0)
    iota_c = lax.broadcasted_iota(jnp.int32, (rows, _LANES), 1)
    iota2d = iota_r * _LANES + iota_c
    lane8 = lax.broadcasted_iota(jnp.int32, (1, 8), 1)
    lane_l = lax.broadcasted_iota(jnp.int32, (1, _LANES), 1)

    def step(t, carry):
        s = s_ref[...]
        m = jnp.max(s)
        idx = jnp.min(jnp.where(s == m, iota2d, rows * _LANES))
        ok = m > -jnp.inf
        r = idx // _LANES
        c = idx % _LANES
        lm = lane_l == c

        def ext(ref):
            return jnp.sum(jnp.where(lm, ref[pl.ds(r, 1), :], 0.0))

        b_ox1 = ext(ox1_ref)
        b_oy1 = ext(oy1_ref)
        b_ox2 = ext(ox2_ref)
        b_oy2 = ext(oy2_ref)
        b_cls = ext(cls_ref)
        b_nx1 = b_ox1 / img_max + b_cls
        b_ny1 = b_oy1 / img_max + b_cls
        b_nx2 = b_ox2 / img_max + b_cls
        b_ny2 = b_oy2 / img_max + b_cls
        a1 = jnp.maximum(b_nx2 - b_nx1, 0.0) * jnp.maximum(b_ny2 - b_ny1, 0.0)

        ltx = jnp.maximum(b_nx1, nx1_ref[...])
        lty = jnp.maximum(b_ny1, ny1_ref[...])
        rbx = jnp.minimum(b_nx2, nx2_ref[...])
        rby = jnp.minimum(b_ny2, ny2_ref[...])
        inter = jnp.maximum(rbx - ltx, 0.0) * jnp.maximum(rby - lty, 0.0)
        iou = inter / (a1 + a2_ref[...] - inter + 1e-7)
        kill = (iou > _IOU_THR) | (iota2d == idx)
        s_ref[...] = jnp.where(kill, -jnp.inf, s)

        okf = jnp.where(ok, 1.0, 0.0)
        sval = jnp.where(ok, m, 0.0)
        row = jnp.where(lane8 == 0, b_ox1,
              jnp.where(lane8 == 1, b_oy1,
              jnp.where(lane8 == 2, b_ox2,
              jnp.where(lane8 == 3, b_oy2,
              jnp.where(lane8 == 4, sval,
              jnp.where(lane8 == 5, b_cls, 0.0)))))) * okf
        out_ref[0, pl.ds(t, 1), :] = row
        return carry

    lax.fori_loop(0, _MAX_DET, step, 0)


def _make_call(B, C, rows, img_max, nc):
    body = functools.partial(_nms_body, img_max=img_max, nc=nc, rows=rows)
    return pl.pallas_call(
        body,
        grid=(B,),
        in_specs=[pl.BlockSpec((1, C, rows, _LANES), lambda b: (b, 0, 0, 0))],
        out_specs=pl.BlockSpec((1, _MD_PAD, 8), lambda b: (b, 0, 0)),
        out_shape=jax.ShapeDtypeStruct((B, _MD_PAD, 8), jnp.float32),
        scratch_shapes=[pltpu.VMEM((rows, _LANES), jnp.float32)] * 11,
    )


@jax.jit
def kernel(x, pred):
    img_max = float(max(x.shape[2], x.shape[3]))
    B, C, A = pred.shape
    nc = C - 4
    ap = ((A + 1023) // 1024) * 1024
    rows = ap // _LANES
    predp = jnp.pad(pred, ((0, 0), (0, 0), (0, ap - A)))
    predp = predp.reshape(B, C, rows, _LANES)
    out = _make_call(B, C, rows, img_max, nc)(predp)
    return out[:, :_MAX_DET, :6]


# speedup vs baseline: 13.4256x; 1.2239x over previous
import functools

import jax
import jax.numpy as jnp
from jax import lax
from jax.experimental import pallas as pl
from jax.experimental.pallas import tpu as pltpu

_CONF = 0.25
_IOU_THR = 0.45
_MAX_DET = 300
_LANES = 128
_MD_PAD = 304
_NPLANES = 11
_IMGS = 8


def _prep_body(pred_ref, out_ref, *, img_max, nc, rows):
    score = pred_ref[0, 4]
    for c in range(1, nc):
        score = jnp.maximum(score, pred_ref[0, 4 + c])
    clsi = jnp.full((rows, _LANES), nc, jnp.int32)
    for c in range(nc):
        clsi = jnp.minimum(clsi, jnp.where(pred_ref[0, 4 + c] == score, c, nc))
    clsf = clsi.astype(jnp.float32)

    bx = pred_ref[0, 0]
    by = pred_ref[0, 1]
    hw = pred_ref[0, 2] / 2.0
    hh = pred_ref[0, 3] / 2.0
    x1 = bx - hw
    y1 = by - hh
    x2 = bx + hw
    y2 = by + hh
    nx1 = x1 / img_max + clsf
    ny1 = y1 / img_max + clsf
    nx2 = x2 / img_max + clsf
    ny2 = y2 / img_max + clsf
    a2 = jnp.maximum(nx2 - nx1, 0.0) * jnp.maximum(ny2 - ny1, 0.0)

    out_ref[0, 0] = jnp.where(score > _CONF, score, -jnp.inf)
    out_ref[0, 1] = x1
    out_ref[0, 2] = y1
    out_ref[0, 3] = x2
    out_ref[0, 4] = y2
    out_ref[0, 5] = nx1
    out_ref[0, 6] = ny1
    out_ref[0, 7] = nx2
    out_ref[0, 8] = ny2
    out_ref[0, 9] = clsf
    out_ref[0, 10] = a2


def _loop_body(pl_ref, out_ref, s_ref, *, img_max, rows):
    for i in range(_IMGS):
        s_ref[i] = pl_ref[0, i, 0]

    iota_r = lax.broadcasted_iota(jnp.int32, (rows, _LANES), 0)
    iota_c = lax.broadcasted_iota(jnp.int32, (rows, _LANES), 1)
    iota2d = iota_r * _LANES + iota_c
    lane8 = lax.broadcasted_iota(jnp.int32, (1, 8), 1)
    lane_l = lax.broadcasted_iota(jnp.int32, (1, _LANES), 1)
    neg_inf = jnp.float32(-jnp.inf)

    def one_image(i, t):
        s = s_ref[i]
        m = jnp.max(s, keepdims=True).reshape(1, 1)
        idx = jnp.min(jnp.where(s == m, iota2d, rows * _LANES))
        r = idx // _LANES
        c = idx % _LANES
        lm = lane_l == c

        def ext(k):
            return jnp.sum(jnp.where(lm, pl_ref[0, i, k, pl.ds(r, 1), :], 0.0),
                           keepdims=True).reshape(1, 1)

        b_ox1 = ext(1)
        b_oy1 = ext(2)
        b_ox2 = ext(3)
        b_oy2 = ext(4)
        b_cls = ext(9)
        b_nx1 = b_ox1 / img_max + b_cls
        b_ny1 = b_oy1 / img_max + b_cls
        b_nx2 = b_ox2 / img_max + b_cls
        b_ny2 = b_oy2 / img_max + b_cls
        a1 = jnp.maximum(b_nx2 - b_nx1, 0.0) * jnp.maximum(b_ny2 - b_ny1, 0.0)

        ltx = jnp.maximum(b_nx1, pl_ref[0, i, 5])
        lty = jnp.maximum(b_ny1, pl_ref[0, i, 6])
        rbx = jnp.minimum(b_nx2, pl_ref[0, i, 7])
        rby = jnp.minimum(b_ny2, pl_ref[0, i, 8])
        inter = jnp.maximum(rbx - ltx, 0.0) * jnp.maximum(rby - lty, 0.0)
        iou = inter / (a1 + pl_ref[0, i, 10] - inter + 1e-7)
        kill = (iou > _IOU_THR) | (iota2d == idx)
        s_ref[i] = jnp.where(kill, neg_inf, s)

        ok = m > neg_inf
        okf = jnp.where(ok, 1.0, 0.0)
        sval = jnp.where(ok, m, 0.0)
        row = jnp.where(lane8 == 0, b_ox1,
              jnp.where(lane8 == 1, b_oy1,
              jnp.where(lane8 == 2, b_ox2,
              jnp.where(lane8 == 3, b_oy2,
              jnp.where(lane8 == 4, sval,
              jnp.where(lane8 == 5, b_cls, 0.0)))))) * okf
        out_ref[0, i, pl.ds(t, 1), :] = row

    def step(t, carry):
        for i in range(_IMGS):
            one_image(i, t)
        return carry

    lax.fori_loop(0, _MAX_DET, step, 0)


def _make_calls(B, C, rows, img_max, nc):
    prep = pl.pallas_call(
        functools.partial(_prep_body, img_max=img_max, nc=nc, rows=rows),
        grid=(B,),
        in_specs=[pl.BlockSpec((1, C, rows, _LANES), lambda b: (b, 0, 0, 0))],
        out_specs=pl.BlockSpec((1, _NPLANES, rows, _LANES), lambda b: (b, 0, 0, 0)),
        out_shape=jax.ShapeDtypeStruct((B, _NPLANES, rows, _LANES), jnp.float32),
    )
    G = B // _IMGS
    loop = pl.pallas_call(
        functools.partial(_loop_body, img_max=img_max, rows=rows),
        grid=(G,),
        in_specs=[pl.BlockSpec((1, _IMGS, _NPLANES, rows, _LANES),
                               lambda g: (g, 0, 0, 0, 0))],
        out_specs=pl.BlockSpec((1, _IMGS, _MD_PAD, 8), lambda g: (g, 0, 0, 0)),
        out_shape=jax.ShapeDtypeStruct((G, _IMGS, _MD_PAD, 8), jnp.float32),
        scratch_shapes=[pltpu.VMEM((_IMGS, rows, _LANES), jnp.float32)],
    )
    return prep, loop


@jax.jit
def kernel(x, pred):
    img_max = float(max(x.shape[2], x.shape[3]))
    B, C, A = pred.shape
    nc = C - 4
    ap = ((A + 1023) // 1024) * 1024
    rows = ap // _LANES
    predp = jnp.pad(pred, ((0, 0), (0, 0), (0, ap - A)))
    predp = predp.reshape(B, C, rows, _LANES)
    prep, loop = _make_calls(B, C, rows, img_max, nc)
    planes = prep(predp)
    G = B // _IMGS
    out = loop(planes.reshape(G, _IMGS, _NPLANES, rows, _LANES))
    return out.reshape(B, _MD_PAD, 8)[:, :_MAX_DET, :6]


# speedup vs baseline: 21.5777x; 1.6072x over previous
import functools

import jax
import jax.numpy as jnp
from jax import lax
from jax.experimental import pallas as pl
from jax.experimental.pallas import tpu as pltpu

_CONF = 0.25
_IOU_THR = 0.45
_MAX_DET = 300
_LANES = 128
_MD_PAD = 304
_NPLANES = 11


def _prep_body(pred_ref, out_ref, *, img_max, nc, rows):
    score = pred_ref[0, 4]
    for c in range(1, nc):
        score = jnp.maximum(score, pred_ref[0, 4 + c])
    clsi = jnp.full((rows, _LANES), nc, jnp.int32)
    for c in range(nc):
        clsi = jnp.minimum(clsi, jnp.where(pred_ref[0, 4 + c] == score, c, nc))
    clsf = clsi.astype(jnp.float32)

    bx = pred_ref[0, 0]
    by = pred_ref[0, 1]
    hw = pred_ref[0, 2] / 2.0
    hh = pred_ref[0, 3] / 2.0
    x1 = bx - hw
    y1 = by - hh
    x2 = bx + hw
    y2 = by + hh
    nx1 = x1 / img_max + clsf
    ny1 = y1 / img_max + clsf
    nx2 = x2 / img_max + clsf
    ny2 = y2 / img_max + clsf
    a2 = jnp.maximum(nx2 - nx1, 0.0) * jnp.maximum(ny2 - ny1, 0.0)

    out_ref[0, 0] = jnp.where(score > _CONF, score, -jnp.inf)
    out_ref[0, 1] = x1
    out_ref[0, 2] = y1
    out_ref[0, 3] = x2
    out_ref[0, 4] = y2
    out_ref[0, 5] = nx1
    out_ref[0, 6] = ny1
    out_ref[0, 7] = nx2
    out_ref[0, 8] = ny2
    out_ref[0, 9] = clsf
    out_ref[0, 10] = a2


def _red(op, a):
    return op(op(a, axis=2, keepdims=True), axis=1, keepdims=True)


def _loop_body(pl_ref, out_ref, s_ref, *, img_max, rows):
    s_ref[...] = pl_ref[0, :, 0]

    iota3 = (lax.broadcasted_iota(jnp.int32, (1, rows, _LANES), 1) * _LANES
             + lax.broadcasted_iota(jnp.int32, (1, rows, _LANES), 2))
    lane8 = lax.broadcasted_iota(jnp.int32, (1, 1, 8), 2)
    neg_inf = jnp.float32(-jnp.inf)
    big = rows * _LANES

    def step(t, carry):
        s = s_ref[...]
        m = _red(jnp.max, s)
        idxv = _red(jnp.min, jnp.where(s == m, iota3, big))
        selmask = iota3 == idxv

        def ext(k):
            return _red(jnp.sum, jnp.where(selmask, pl_ref[0, :, k], 0.0))

        b_ox1 = ext(1)
        b_oy1 = ext(2)
        b_ox2 = ext(3)
        b_oy2 = ext(4)
        b_cls = ext(9)
        b_nx1 = b_ox1 / img_max + b_cls
        b_ny1 = b_oy1 / img_max + b_cls
        b_nx2 = b_ox2 / img_max + b_cls
        b_ny2 = b_oy2 / img_max + b_cls
        a1 = jnp.maximum(b_nx2 - b_nx1, 0.0) * jnp.maximum(b_ny2 - b_ny1, 0.0)

        ltx = jnp.maximum(b_nx1, pl_ref[0, :, 5])
        lty = jnp.maximum(b_ny1, pl_ref[0, :, 6])
        rbx = jnp.minimum(b_nx2, pl_ref[0, :, 7])
        rby = jnp.minimum(b_ny2, pl_ref[0, :, 8])
        inter = jnp.maximum(rbx - ltx, 0.0) * jnp.maximum(rby - lty, 0.0)
        iou = inter / (a1 + pl_ref[0, :, 10] - inter + 1e-7)
        kill = (iou > _IOU_THR) | selmask
        s_ref[...] = jnp.where(kill, neg_inf, s)

        ok = m > neg_inf
        okf = jnp.where(ok, 1.0, 0.0)
        sval = jnp.where(ok, m, 0.0)
        row = jnp.where(lane8 == 0, b_ox1,
              jnp.where(lane8 == 1, b_oy1,
              jnp.where(lane8 == 2, b_ox2,
              jnp.where(lane8 == 3, b_oy2,
              jnp.where(lane8 == 4, sval,
              jnp.where(lane8 == 5, b_cls, 0.0)))))) * okf
        out_ref[0, :, pl.ds(t, 1), :] = row
        return carry

    lax.fori_loop(0, _MAX_DET, step, 0)


def _make_calls(B, C, rows, img_max, nc):
    prep = pl.pallas_call(
        functools.partial(_prep_body, img_max=img_max, nc=nc, rows=rows),
        grid=(B,),
        in_specs=[pl.BlockSpec((1, C, rows, _LANES), lambda b: (b, 0, 0, 0))],
        out_specs=pl.BlockSpec((1, _NPLANES, rows, _LANES), lambda b: (b, 0, 0, 0)),
        out_shape=jax.ShapeDtypeStruct((B, _NPLANES, rows, _LANES), jnp.float32),
    )
    loop = pl.pallas_call(
        functools.partial(_loop_body, img_max=img_max, rows=rows),
        grid=(1,),
        in_specs=[pl.BlockSpec((1, B, _NPLANES, rows, _LANES),
                               lambda g: (g, 0, 0, 0, 0))],
        out_specs=pl.BlockSpec((1, B, _MD_PAD, 8), lambda g: (g, 0, 0, 0)),
        out_shape=jax.ShapeDtypeStruct((1, B, _MD_PAD, 8), jnp.float32),
        scratch_shapes=[pltpu.VMEM((B, rows, _LANES), jnp.float32)],
    )
    return prep, loop


@jax.jit
def kernel(x, pred):
    img_max = float(max(x.shape[2], x.shape[3]))
    B, C, A = pred.shape
    nc = C - 4
    ap = ((A + 1023) // 1024) * 1024
    rows = ap // _LANES
    predp = jnp.pad(pred, ((0, 0), (0, 0), (0, ap - A)))
    predp = predp.reshape(B, C, rows, _LANES)
    prep, loop = _make_calls(B, C, rows, img_max, nc)
    planes = prep(predp)
    out = loop(planes.reshape(1, B, _NPLANES, rows, _LANES))
    return out.reshape(B, _MD_PAD, 8)[:, :_MAX_DET, :6]


# speedup vs baseline: 30.3290x; 1.4056x over previous
import functools

import jax
import jax.numpy as jnp
from jax import lax
from jax.experimental import pallas as pl
from jax.experimental.pallas import tpu as pltpu

_CONF = 0.25
_IOU_THR = 0.45
_MAX_DET = 300
_LANES = 128
_MD_PAD = 304
_NPLANES = 11
_GBATCH = 8


def _prep_body(pred_ref, out_ref, *, img_max, nc, rows):
    score = pred_ref[0, 4]
    for c in range(1, nc):
        score = jnp.maximum(score, pred_ref[0, 4 + c])
    clsi = jnp.full((rows, _LANES), nc, jnp.int32)
    for c in range(nc):
        clsi = jnp.minimum(clsi, jnp.where(pred_ref[0, 4 + c] == score, c, nc))
    clsf = clsi.astype(jnp.float32)

    bx = pred_ref[0, 0]
    by = pred_ref[0, 1]
    hw = pred_ref[0, 2] / 2.0
    hh = pred_ref[0, 3] / 2.0
    x1 = bx - hw
    y1 = by - hh
    x2 = bx + hw
    y2 = by + hh
    nx1 = x1 / img_max + clsf
    ny1 = y1 / img_max + clsf
    nx2 = x2 / img_max + clsf
    ny2 = y2 / img_max + clsf
    a2 = jnp.maximum(nx2 - nx1, 0.0) * jnp.maximum(ny2 - ny1, 0.0)

    out_ref[0, 0] = jnp.where(score > _CONF, score, -jnp.inf)
    out_ref[0, 1] = x1
    out_ref[0, 2] = y1
    out_ref[0, 3] = x2
    out_ref[0, 4] = y2
    out_ref[0, 5] = nx1
    out_ref[0, 6] = ny1
    out_ref[0, 7] = nx2
    out_ref[0, 8] = ny2
    out_ref[0, 9] = clsf
    out_ref[0, 10] = a2


def _red(op, a):
    return op(op(a, axis=1, keepdims=True), axis=2, keepdims=True)


def _loop_body(pl_ref, out_ref, s_ref, *, img_max, rows):
    s_ref[...] = pl_ref[0, :, 0]

    iota3 = (lax.broadcasted_iota(jnp.int32, (1, rows, _LANES), 1) * _LANES
             + lax.broadcasted_iota(jnp.int32, (1, rows, _LANES), 2))
    lane8 = lax.broadcasted_iota(jnp.int32, (1, 1, 8), 2)
    neg_inf = jnp.float32(-jnp.inf)
    big = rows * _LANES

    def step(t, carry):
        s = s_ref[...]
        m = _red(jnp.max, s)
        idxv = _red(jnp.min, jnp.where(s == m, iota3, big))
        selmask = iota3 == idxv

        def ext(k):
            return _red(jnp.sum, jnp.where(selmask, pl_ref[0, :, k], 0.0))

        b_ox1 = ext(1)
        b_oy1 = ext(2)
        b_ox2 = ext(3)
        b_oy2 = ext(4)
        b_cls = ext(9)
        b_nx1 = b_ox1 / img_max + b_cls
        b_ny1 = b_oy1 / img_max + b_cls
        b_nx2 = b_ox2 / img_max + b_cls
        b_ny2 = b_oy2 / img_max + b_cls
        a1 = jnp.maximum(b_nx2 - b_nx1, 0.0) * jnp.maximum(b_ny2 - b_ny1, 0.0)

        ltx = jnp.maximum(b_nx1, pl_ref[0, :, 5])
        lty = jnp.maximum(b_ny1, pl_ref[0, :, 6])
        rbx = jnp.minimum(b_nx2, pl_ref[0, :, 7])
        rby = jnp.minimum(b_ny2, pl_ref[0, :, 8])
        inter = jnp.maximum(rbx - ltx, 0.0) * jnp.maximum(rby - lty, 0.0)
        iou = inter / (a1 + pl_ref[0, :, 10] - inter + 1e-7)
        kill = (iou > _IOU_THR) | selmask
        s_ref[...] = jnp.where(kill, neg_inf, s)

        ok = m > neg_inf
        okf = jnp.where(ok, 1.0, 0.0)
        sval = jnp.where(ok, m, 0.0)
        row = jnp.where(lane8 == 0, b_ox1,
              jnp.where(lane8 == 1, b_oy1,
              jnp.where(lane8 == 2, b_ox2,
              jnp.where(lane8 == 3, b_oy2,
              jnp.where(lane8 == 4, sval,
              jnp.where(lane8 == 5, b_cls, 0.0)))))) * okf
        out_ref[0, :, pl.ds(t, 1), :] = row
        return carry

    lax.fori_loop(0, _MAX_DET, step, 0)


def _make_calls(B, C, rows, img_max, nc):
    prep = pl.pallas_call(
        functools.partial(_prep_body, img_max=img_max, nc=nc, rows=rows),
        grid=(B,),
        in_specs=[pl.BlockSpec((1, C, rows, _LANES), lambda b: (b, 0, 0, 0))],
        out_specs=pl.BlockSpec((1, _NPLANES, rows, _LANES), lambda b: (b, 0, 0, 0)),
        out_shape=jax.ShapeDtypeStruct((B, _NPLANES, rows, _LANES), jnp.float32),
    )
    gb = _GBATCH if B % _GBATCH == 0 else B
    loop = pl.pallas_call(
        functools.partial(_loop_body, img_max=img_max, rows=rows),
        grid=(B // gb,),
        in_specs=[pl.BlockSpec((1, gb, _NPLANES, rows, _LANES),
                               lambda g: (g, 0, 0, 0, 0))],
        out_specs=pl.BlockSpec((1, gb, _MD_PAD, 8), lambda g: (g, 0, 0, 0)),
        out_shape=jax.ShapeDtypeStruct((B // gb, gb, _MD_PAD, 8), jnp.float32),
        scratch_shapes=[pltpu.VMEM((gb, rows, _LANES), jnp.float32)],
    )
    return prep, loop, gb


@jax.jit
def kernel(x, pred):
    img_max = float(max(x.shape[2], x.shape[3]))
    B, C, A = pred.shape
    nc = C - 4
    ap = ((A + 1023) // 1024) * 1024
    rows = ap // _LANES
    predp = jnp.pad(pred, ((0, 0), (0, 0), (0, ap - A)))
    predp = predp.reshape(B, C, rows, _LANES)
    prep, loop, gb = _make_calls(B, C, rows, img_max, nc)
    planes = prep(predp)
    out = loop(planes.reshape(B // gb, gb, _NPLANES, rows, _LANES))
    return out.reshape(B, _MD_PAD, 8)[:, :_MAX_DET, :6]


# speedup vs baseline: 32.3256x; 1.0658x over previous
import functools

import jax
import jax.numpy as jnp
from jax import lax
from jax.experimental import pallas as pl
from jax.experimental.pallas import tpu as pltpu

_CONF = 0.25
_IOU_THR = 0.45
_MAX_DET = 300
_LANES = 128
_MD_PAD = 304
_NPLANES = 11
_GBATCH = 16


def _prep_body(pred_ref, out_ref, *, img_max, nc, rows):
    score = pred_ref[0, 4]
    for c in range(1, nc):
        score = jnp.maximum(score, pred_ref[0, 4 + c])
    clsi = jnp.full((rows, _LANES), nc, jnp.int32)
    for c in range(nc):
        clsi = jnp.minimum(clsi, jnp.where(pred_ref[0, 4 + c] == score, c, nc))
    clsf = clsi.astype(jnp.float32)

    bx = pred_ref[0, 0]
    by = pred_ref[0, 1]
    hw = pred_ref[0, 2] / 2.0
    hh = pred_ref[0, 3] / 2.0
    x1 = bx - hw
    y1 = by - hh
    x2 = bx + hw
    y2 = by + hh
    nx1 = x1 / img_max + clsf
    ny1 = y1 / img_max + clsf
    nx2 = x2 / img_max + clsf
    ny2 = y2 / img_max + clsf
    a2 = jnp.maximum(nx2 - nx1, 0.0) * jnp.maximum(ny2 - ny1, 0.0)

    out_ref[0, 0] = jnp.where(score > _CONF, score, -jnp.inf)
    out_ref[0, 1] = x1
    out_ref[0, 2] = y1
    out_ref[0, 3] = x2
    out_ref[0, 4] = y2
    out_ref[0, 5] = nx1
    out_ref[0, 6] = ny1
    out_ref[0, 7] = nx2
    out_ref[0, 8] = ny2
    out_ref[0, 9] = clsf
    out_ref[0, 10] = a2


def _red(op, a):
    return op(op(a, axis=1, keepdims=True), axis=2, keepdims=True)


def _loop_body(pl_ref, out_ref, s_ref, *, img_max, rows):
    s_ref[...] = pl_ref[0, :, 0]

    iota3 = (lax.broadcasted_iota(jnp.int32, (1, rows, _LANES), 1) * _LANES
             + lax.broadcasted_iota(jnp.int32, (1, rows, _LANES), 2))
    lane8 = lax.broadcasted_iota(jnp.int32, (1, 1, 8), 2)
    neg_inf = jnp.float32(-jnp.inf)
    big = rows * _LANES

    def step(t, carry):
        s = s_ref[...]
        m = _red(jnp.max, s)
        idxv = _red(jnp.min, jnp.where(s == m, iota3, big))
        selmask = iota3 == idxv

        def ext(k):
            return _red(jnp.sum, jnp.where(selmask, pl_ref[0, :, k], 0.0))

        b_ox1 = ext(1)
        b_oy1 = ext(2)
        b_ox2 = ext(3)
        b_oy2 = ext(4)
        b_cls = ext(9)
        b_nx1 = b_ox1 / img_max + b_cls
        b_ny1 = b_oy1 / img_max + b_cls
        b_nx2 = b_ox2 / img_max + b_cls
        b_ny2 = b_oy2 / img_max + b_cls
        a1 = jnp.maximum(b_nx2 - b_nx1, 0.0) * jnp.maximum(b_ny2 - b_ny1, 0.0)

        ltx = jnp.maximum(b_nx1, pl_ref[0, :, 5])
        lty = jnp.maximum(b_ny1, pl_ref[0, :, 6])
        rbx = jnp.minimum(b_nx2, pl_ref[0, :, 7])
        rby = jnp.minimum(b_ny2, pl_ref[0, :, 8])
        inter = jnp.maximum(rbx - ltx, 0.0) * jnp.maximum(rby - lty, 0.0)
        iou = inter / (a1 + pl_ref[0, :, 10] - inter + 1e-7)
        kill = (iou > _IOU_THR) | selmask
        s_ref[...] = jnp.where(kill, neg_inf, s)

        ok = m > neg_inf
        okf = jnp.where(ok, 1.0, 0.0)
        sval = jnp.where(ok, m, 0.0)
        row = jnp.where(lane8 == 0, b_ox1,
              jnp.where(lane8 == 1, b_oy1,
              jnp.where(lane8 == 2, b_ox2,
              jnp.where(lane8 == 3, b_oy2,
              jnp.where(lane8 == 4, sval,
              jnp.where(lane8 == 5, b_cls, 0.0)))))) * okf
        out_ref[0, :, pl.ds(t, 1), :] = row
        return carry

    lax.fori_loop(0, _MAX_DET, step, 0)


def _make_calls(B, C, rows, img_max, nc):
    prep = pl.pallas_call(
        functools.partial(_prep_body, img_max=img_max, nc=nc, rows=rows),
        grid=(B,),
        in_specs=[pl.BlockSpec((1, C, rows, _LANES), lambda b: (b, 0, 0, 0))],
        out_specs=pl.BlockSpec((1, _NPLANES, rows, _LANES), lambda b: (b, 0, 0, 0)),
        out_shape=jax.ShapeDtypeStruct((B, _NPLANES, rows, _LANES), jnp.float32),
    )
    gb = _GBATCH if B % _GBATCH == 0 else B
    loop = pl.pallas_call(
        functools.partial(_loop_body, img_max=img_max, rows=rows),
        grid=(B // gb,),
        in_specs=[pl.BlockSpec((1, gb, _NPLANES, rows, _LANES),
                               lambda g: (g, 0, 0, 0, 0))],
        out_specs=pl.BlockSpec((1, gb, _MD_PAD, 8), lambda g: (g, 0, 0, 0)),
        out_shape=jax.ShapeDtypeStruct((B // gb, gb, _MD_PAD, 8), jnp.float32),
        scratch_shapes=[pltpu.VMEM((gb, rows, _LANES), jnp.float32)],
    )
    return prep, loop, gb


@jax.jit
def kernel(x, pred):
    img_max = float(max(x.shape[2], x.shape[3]))
    B, C, A = pred.shape
    nc = C - 4
    ap = ((A + 1023) // 1024) * 1024
    rows = ap // _LANES
    predp = jnp.pad(pred, ((0, 0), (0, 0), (0, ap - A)))
    predp = predp.reshape(B, C, rows, _LANES)
    prep, loop, gb = _make_calls(B, C, rows, img_max, nc)
    planes = prep(predp)
    out = loop(planes.reshape(B // gb, gb, _NPLANES, rows, _LANES))
    return out.reshape(B, _MD_PAD, 8)[:, :_MAX_DET, :6]
